# Initial kernel scaffold; baseline (speedup 1.0000x reference)
import jax
import jax.numpy as jnp
from jax.experimental import pallas as pl


def kernel(obs, enc_W, enc_b, dec_W, dec_b, codebook):
    raise NotImplementedError("write your pallas kernel here")


# speedup vs baseline: 1.3263x; 1.3263x over previous
import functools

import jax
import jax.numpy as jnp
from jax import lax
from jax.experimental import pallas as pl
from jax.experimental.pallas import tpu as pltpu
from jax.experimental.pallas import tpu_sc as plsc

VOCAB = 8192
EMBED = 256
NTOK = 16
OBS_DIM = 128

_HI = lax.Precision.HIGHEST


_ENC_MB = 512


def _enc_body(obs_ref, w_ref, b_ref, out_ref):
    out_ref[...] = (
        jnp.dot(obs_ref[...], w_ref[...], preferred_element_type=jnp.float32,
                precision=_HI)
        + b_ref[...])


def _encode(obs, enc_W, enc_b):
    b = obs.shape[0]
    n = NTOK * EMBED
    return pl.pallas_call(
        _enc_body,
        grid=(b // _ENC_MB,),
        in_specs=[
            pl.BlockSpec((_ENC_MB, OBS_DIM), lambda i: (i, 0)),
            pl.BlockSpec((OBS_DIM, n), lambda i: (0, 0)),
            pl.BlockSpec((1, n), lambda i: (0, 0)),
        ],
        out_specs=pl.BlockSpec((_ENC_MB, n), lambda i: (i, 0)),
        out_shape=jax.ShapeDtypeStruct((b, n), jnp.float32),
    )(obs, enc_W, enc_b.reshape(1, n))


_RB = 2048
_VB = 2048


def _argmin_body(z_ref, cb_ref, out_ref, rmin_ref, rarg_ref):
    v = pl.program_id(1)
    z = z_ref[...]
    cb = cb_ref[...]
    a = jnp.sum(z * z, axis=1, keepdims=True)
    bn = jnp.sum(cb * cb, axis=1, keepdims=True)
    zb = (2.0 * z).astype(jnp.bfloat16)
    cbb = cb.astype(jnp.bfloat16)
    m = lax.dot_general(zb, cbb, (((1,), (1,)), ((), ())),
                        preferred_element_type=jnp.float32)
    dist = (a + bn.T) - m
    tmin = jnp.min(dist, axis=1, keepdims=True)
    jidx = lax.broadcasted_iota(jnp.int32, dist.shape, 1)
    targ = jnp.min(jnp.where(dist == tmin, jidx, VOCAB), axis=1,
                   keepdims=True) + v * _VB

    @pl.when(v == 0)
    def _():
        rmin_ref[...] = tmin
        rarg_ref[...] = targ

    @pl.when(v > 0)
    def _():
        upd = tmin < rmin_ref[...]
        rmin_ref[...] = jnp.where(upd, tmin, rmin_ref[...])
        rarg_ref[...] = jnp.where(upd, targ, rarg_ref[...])

    @pl.when(v == pl.num_programs(1) - 1)
    def _():
        out_ref[...] = rarg_ref[...]


def _argmin_tokens(z_flat, codebook):
    rows = z_flat.shape[0]
    return pl.pallas_call(
        _argmin_body,
        grid=(rows // _RB, VOCAB // _VB),
        in_specs=[
            pl.BlockSpec((_RB, EMBED), lambda r, v: (r, 0)),
            pl.BlockSpec((_VB, EMBED), lambda r, v: (v, 0)),
        ],
        out_specs=pl.BlockSpec((_RB, 1), lambda r, v: (r, 0)),
        out_shape=jax.ShapeDtypeStruct((rows, 1), jnp.int32),
        scratch_shapes=[
            pltpu.VMEM((_RB, 1), jnp.float32),
            pltpu.VMEM((_RB, 1), jnp.int32),
        ],
    )(z_flat, codebook)


_NW = 32
_CH = 256


def _sc_gather(codebook, tokens3):
    nw, nch, ch = tokens3.shape
    rows = nw * nch * ch
    mesh = plsc.VectorSubcoreMesh(core_axis_name="c", subcore_axis_name="s")

    @functools.partial(
        pl.kernel,
        out_type=jax.ShapeDtypeStruct((rows, EMBED), jnp.float32),
        mesh=mesh,
        scratch_types=[
            pltpu.VMEM((nch, ch), jnp.int32),
            pltpu.VMEM((ch, EMBED), jnp.float32),
            pltpu.SemaphoreType.DMA,
        ],
        compiler_params=pltpu.CompilerParams(use_tc_tiling_on_sc=False),
    )
    def k(cb_hbm, idx_hbm, out_hbm, idx_v, rows_v, sem):
        wid = lax.axis_index("s") * 2 + lax.axis_index("c")
        pltpu.sync_copy(idx_hbm.at[wid], idx_v)

        def body(c, carry):
            pltpu.async_copy(cb_hbm.at[idx_v.at[c]], rows_v, sem).wait()
            base = wid * (nch * ch) + c * ch
            pltpu.sync_copy(rows_v, out_hbm.at[pl.ds(base, ch)])
            return carry

        lax.fori_loop(0, nch, body, 0)

    return k(codebook, tokens3)


_DEC_MB = 512


def _dec_body(zq_ref, w_ref, b_ref, out_ref):
    out_ref[...] = (
        jnp.dot(zq_ref[...], w_ref[...], preferred_element_type=jnp.float32,
                precision=_HI)
        + b_ref[...])


def _decode(zq_flat, dec_W, dec_b):
    b = zq_flat.shape[0]
    n = NTOK * EMBED
    return pl.pallas_call(
        _dec_body,
        grid=(b // _DEC_MB,),
        in_specs=[
            pl.BlockSpec((_DEC_MB, n), lambda i: (i, 0)),
            pl.BlockSpec((n, OBS_DIM), lambda i: (0, 0)),
            pl.BlockSpec((1, OBS_DIM), lambda i: (0, 0)),
        ],
        out_specs=pl.BlockSpec((_DEC_MB, OBS_DIM), lambda i: (i, 0)),
        out_shape=jax.ShapeDtypeStruct((b, OBS_DIM), jnp.float32),
    )(zq_flat, dec_W, dec_b.reshape(1, OBS_DIM))


def kernel(obs, enc_W, enc_b, dec_W, dec_b, codebook):
    b = obs.shape[0]
    rows = b * NTOK
    z2 = _encode(obs, enc_W, enc_b)
    z_flat = z2.reshape(rows, EMBED)
    tokens = _argmin_tokens(z_flat, codebook)
    tokens3 = tokens.reshape(_NW, rows // (_NW * _CH), _CH)
    zq_flat = _sc_gather(codebook, tokens3)
    rec = _decode(zq_flat.reshape(b, NTOK * EMBED), dec_W, dec_b)
    z = z2.reshape(b, NTOK, EMBED)
    z_q = zq_flat.reshape(b, NTOK, EMBED)
    return z, z_q, rec


# speedup vs baseline: 1.4576x; 1.0990x over previous
import functools

import jax
import jax.numpy as jnp
from jax import lax
from jax.experimental import pallas as pl
from jax.experimental.pallas import tpu as pltpu
from jax.experimental.pallas import tpu_sc as plsc

VOCAB = 8192
EMBED = 256
NTOK = 16
OBS_DIM = 128

_HI = lax.Precision.HIGHEST


_ENC_MB = 512


def _enc_body(obs_ref, w_ref, b_ref, out_ref):
    out_ref[...] = (
        jnp.dot(obs_ref[...], w_ref[...], preferred_element_type=jnp.float32,
                precision=_HI)
        + b_ref[...])


def _encode(obs, enc_W, enc_b):
    b = obs.shape[0]
    n = NTOK * EMBED
    return pl.pallas_call(
        _enc_body,
        grid=(b // _ENC_MB,),
        in_specs=[
            pl.BlockSpec((_ENC_MB, OBS_DIM), lambda i: (i, 0)),
            pl.BlockSpec((OBS_DIM, n), lambda i: (0, 0)),
            pl.BlockSpec((1, n), lambda i: (0, 0)),
        ],
        out_specs=pl.BlockSpec((_ENC_MB, n), lambda i: (i, 0)),
        out_shape=jax.ShapeDtypeStruct((b, n), jnp.float32),
    )(obs, enc_W, enc_b.reshape(1, n))


_RB = 2048
_VB = 2048


def _argmin_body(z_ref, cb_ref, out_ref, rmin_ref, rarg_ref):
    v = pl.program_id(1)
    z = z_ref[...]
    cb = cb_ref[...]
    bn = jnp.sum(cb * cb, axis=1, keepdims=True)
    zb = (2.0 * z).astype(jnp.bfloat16)
    cbb = cb.astype(jnp.bfloat16)
    m = lax.dot_general(zb, cbb, (((1,), (1,)), ((), ())),
                        preferred_element_type=jnp.float32)
    dist = bn.T - m
    tmin = jnp.min(dist, axis=1, keepdims=True)
    jidx = lax.broadcasted_iota(jnp.int32, dist.shape, 1)
    targ = jnp.min(jnp.where(dist == tmin, jidx, VOCAB), axis=1,
                   keepdims=True) + v * _VB

    @pl.when(v == 0)
    def _():
        rmin_ref[...] = tmin
        rarg_ref[...] = targ

    @pl.when(v > 0)
    def _():
        upd = tmin < rmin_ref[...]
        rmin_ref[...] = jnp.where(upd, tmin, rmin_ref[...])
        rarg_ref[...] = jnp.where(upd, targ, rarg_ref[...])

    @pl.when(v == pl.num_programs(1) - 1)
    def _():
        out_ref[...] = rarg_ref[...]


def _argmin_tokens(z_flat, codebook):
    rows = z_flat.shape[0]
    return pl.pallas_call(
        _argmin_body,
        grid=(rows // _RB, VOCAB // _VB),
        in_specs=[
            pl.BlockSpec((_RB, EMBED), lambda r, v: (r, 0)),
            pl.BlockSpec((_VB, EMBED), lambda r, v: (v, 0)),
        ],
        out_specs=pl.BlockSpec((_RB, 1), lambda r, v: (r, 0)),
        out_shape=jax.ShapeDtypeStruct((rows, 1), jnp.int32),
        scratch_shapes=[
            pltpu.VMEM((_RB, 1), jnp.float32),
            pltpu.VMEM((_RB, 1), jnp.int32),
        ],
    )(z_flat, codebook)


_NW = 32
_CH = 128


def _sc_gather(codebook, tokens3):
    nw, nch, ch = tokens3.shape
    rows = nw * nch * ch
    mesh = plsc.VectorSubcoreMesh(core_axis_name="c", subcore_axis_name="s")

    @functools.partial(
        pl.kernel,
        out_type=jax.ShapeDtypeStruct((rows, EMBED), jnp.float32),
        mesh=mesh,
        scratch_types=[
            pltpu.VMEM((nch, ch), jnp.int32),
            pltpu.VMEM((ch, EMBED), jnp.float32),
            pltpu.VMEM((ch, EMBED), jnp.float32),
            pltpu.SemaphoreType.DMA,
            pltpu.SemaphoreType.DMA,
        ],
        compiler_params=pltpu.CompilerParams(use_tc_tiling_on_sc=False),
    )
    def k(cb_hbm, idx_hbm, out_hbm, idx_v, rows0_v, rows1_v, sem0, sem1):
        wid = lax.axis_index("s") * 2 + lax.axis_index("c")
        base = wid * (nch * ch)
        pltpu.sync_copy(idx_hbm.at[wid], idx_v)

        pltpu.async_copy(cb_hbm.at[idx_v.at[0]], rows0_v, sem0)

        def body(c, carry):
            even = c % 2 == 0
            nxt = c + 1

            @pl.when(jnp.logical_and(nxt < nch, even))
            def _():
                pltpu.async_copy(cb_hbm.at[idx_v.at[nxt]], rows1_v, sem1)

            @pl.when(jnp.logical_and(nxt < nch, jnp.logical_not(even)))
            def _():
                pltpu.async_copy(cb_hbm.at[idx_v.at[nxt]], rows0_v, sem0)

            @pl.when(even)
            def _():
                pltpu.make_async_copy(cb_hbm.at[idx_v.at[0]], rows0_v, sem0).wait()
                pltpu.sync_copy(rows0_v, out_hbm.at[pl.ds(base + c * ch, ch)])

            @pl.when(jnp.logical_not(even))
            def _():
                pltpu.make_async_copy(cb_hbm.at[idx_v.at[0]], rows1_v, sem1).wait()
                pltpu.sync_copy(rows1_v, out_hbm.at[pl.ds(base + c * ch, ch)])

            return carry

        lax.fori_loop(0, nch, body, 0)

    return k(codebook, tokens3)


_DEC_MB = 512


def _dec_body(zq_ref, w_ref, b_ref, out_ref):
    out_ref[...] = (
        jnp.dot(zq_ref[...], w_ref[...], preferred_element_type=jnp.float32,
                precision=_HI)
        + b_ref[...])


def _decode(zq_flat, dec_W, dec_b):
    b = zq_flat.shape[0]
    n = NTOK * EMBED
    return pl.pallas_call(
        _dec_body,
        grid=(b // _DEC_MB,),
        in_specs=[
            pl.BlockSpec((_DEC_MB, n), lambda i: (i, 0)),
            pl.BlockSpec((n, OBS_DIM), lambda i: (0, 0)),
            pl.BlockSpec((1, OBS_DIM), lambda i: (0, 0)),
        ],
        out_specs=pl.BlockSpec((_DEC_MB, OBS_DIM), lambda i: (i, 0)),
        out_shape=jax.ShapeDtypeStruct((b, OBS_DIM), jnp.float32),
    )(zq_flat, dec_W, dec_b.reshape(1, OBS_DIM))


def kernel(obs, enc_W, enc_b, dec_W, dec_b, codebook):
    b = obs.shape[0]
    rows = b * NTOK
    z2 = _encode(obs, enc_W, enc_b)
    z_flat = z2.reshape(rows, EMBED)
    tokens = _argmin_tokens(z_flat, codebook)
    tokens3 = tokens.reshape(_NW, rows // (_NW * _CH), _CH)
    zq_flat = _sc_gather(codebook, tokens3)
    rec = _decode(zq_flat.reshape(b, NTOK * EMBED), dec_W, dec_b)
    z = z2.reshape(b, NTOK, EMBED)
    z_q = zq_flat.reshape(b, NTOK, EMBED)
    return z, z_q, rec
